# baseline (speedup 1.0000x reference)
import jax
import jax.numpy as jnp
from jax import lax
from jax.experimental import pallas as pl
from jax.experimental.pallas import tpu as pltpu
from jax.experimental.pallas import tpu_sc as plsc

NUM_VOCAB = 1000000
EMBED_DIM = 32
BS = 4096
FEAT_LEN = 26
WORD_LEN = 20

L = 16
NC, NS = 2, 16
NW = NC * NS
ROWS = BS * FEAT_LEN
BATCH_PER_W = BS // NW
CHUNK_B = 2
CHUNK = CHUNK_B * FEAT_LEN
N_CHUNKS = BATCH_PER_W // CHUNK_B
IDX_PER_CHUNK = CHUNK * WORD_LEN
GATHER = 104
N_GATHER = IDX_PER_CHUNK // GATHER
OPAD = 128


def _embed_mean_kernel(idx_hbm, table_hbm, out_hbm,
                       idx_a, idx_b, rows_a, rows_b, out_v, sem_a, sem_b):
    wid = lax.axis_index("s") * NC + lax.axis_index("c")
    inv_n = jnp.float32(1.0 / WORD_LEN)

    def stage(c, idx_v, rows_v, sem):
        gbase = (wid * N_CHUNKS + c) * IDX_PER_CHUNK
        pltpu.sync_copy(idx_hbm.at[pl.ds(gbase, IDX_PER_CHUNK)], idx_v)
        for j in range(N_GATHER):
            pltpu.async_copy(
                table_hbm.at[idx_v.at[pl.ds(j * GATHER, GATHER)]],
                rows_v.at[pl.ds(j * GATHER, GATHER)],
                sem,
            )

    def drain(idx_v, rows_v, sem):
        for j in range(N_GATHER):
            pltpu.make_async_copy(
                table_hbm.at[idx_v.at[pl.ds(j * GATHER, GATHER)]],
                rows_v.at[pl.ds(j * GATHER, GATHER)],
                sem,
            ).wait()

    def compute(c, rows_v):
        def row_body(r, carry):
            base = r * WORD_LEN
            acc0 = rows_v[base, pl.ds(0, L)]
            acc1 = rows_v[base, pl.ds(L, L)]
            for j in range(1, WORD_LEN):
                acc0 = acc0 + rows_v[base + j, pl.ds(0, L)]
                acc1 = acc1 + rows_v[base + j, pl.ds(L, L)]
            out_v[r, pl.ds(0, L)] = acc0 * inv_n
            out_v[r, pl.ds(L, L)] = acc1 * inv_n
            return carry

        lax.fori_loop(0, CHUNK, row_body, 0, unroll=2)
        b0 = wid * BATCH_PER_W + c * CHUNK_B
        for k in range(CHUNK_B):
            pltpu.sync_copy(
                out_v.at[pl.ds(k * FEAT_LEN, FEAT_LEN)],
                out_hbm.at[pl.ds((b0 + k) * 32, FEAT_LEN), pl.ds(0, EMBED_DIM)],
            )

    stage(0, idx_a, rows_a, sem_a)

    def pair_body(i, carry):
        c0 = 2 * i
        c1 = 2 * i + 1
        stage(c1, idx_b, rows_b, sem_b)
        drain(idx_a, rows_a, sem_a)
        compute(c0, rows_a)
        stage((c0 + 2) % N_CHUNKS, idx_a, rows_a, sem_a)
        drain(idx_b, rows_b, sem_b)
        compute(c1, rows_b)
        return carry

    lax.fori_loop(0, N_CHUNKS // 2, pair_body, 0, unroll=False)
    drain(idx_a, rows_a, sem_a)


CBLK = 8192
NBLK = NUM_VOCAB // CBLK
MAIN = NBLK * CBLK
TAIL = NUM_VOCAB - MAIN
GROUPS = -(-NBLK // 4)
GRID = GROUPS + 1
OUT_ROWS = GRID * CBLK
VIEW_ROWS = OUT_ROWS * 4
TAIL_Q0 = 4 * GROUPS * CBLK - 4 * MAIN


def _transpose_kernel(t0, t1, t2, t3, tail_ref, out_ref):
    b = pl.program_id(0)

    @pl.when(b < GROUPS)
    def _main():
        stacked = jnp.concatenate([t0[...], t1[...], t2[...], t3[...]], axis=0)
        out_ref[...] = stacked.T

    @pl.when(b == GROUPS)
    def _tail():
        out_ref[0:TAIL, 0:EMBED_DIM] = tail_ref[...]


def _relayout_table(table):
    t = table.T
    tail = jax.lax.slice(table, (MAIN, 0), (NUM_VOCAB, EMBED_DIM))
    specs = [
        pl.BlockSpec((32, CBLK), lambda b, u=u: (0, jnp.minimum(4 * b + u, NBLK - 1)))
        for u in range(4)
    ]
    specs.append(pl.BlockSpec((TAIL, EMBED_DIM), lambda b: (0, 0)))
    return pl.pallas_call(
        _transpose_kernel,
        out_shape=jax.ShapeDtypeStruct((OUT_ROWS, 128), jnp.float32),
        grid=(GRID,),
        in_specs=specs,
        out_specs=pl.BlockSpec((CBLK, 128), lambda b: (b, 0)),
        compiler_params=pltpu.CompilerParams(
            dimension_semantics=("parallel",)),
    )(t, t, t, t, tail)


SLICE_B = 32


def _slice_kernel(in_ref, out_ref):
    out_ref[...] = in_ref[:, :FEAT_LEN, :EMBED_DIM]


def _slice_out(padded):
    return pl.pallas_call(
        _slice_kernel,
        out_shape=jax.ShapeDtypeStruct((BS, FEAT_LEN, EMBED_DIM), jnp.float32),
        grid=(BS // SLICE_B,),
        in_specs=[pl.BlockSpec((SLICE_B, 32, OPAD), lambda i: (i, 0, 0))],
        out_specs=pl.BlockSpec((SLICE_B, FEAT_LEN, EMBED_DIM),
                               lambda i: (i, 0, 0)),
        compiler_params=pltpu.CompilerParams(
            dimension_semantics=("parallel",)),
    )(padded)


@jax.jit
def kernel(input_text, table):
    flat = input_text.reshape(ROWS * WORD_LEN).astype(jnp.int32)
    m = flat // CBLK
    q_main = 4 * ((m // 4) * CBLK + (flat % CBLK)) + (m % 4)
    idx = jnp.where(flat >= MAIN, 4 * flat + TAIL_Q0, q_main)
    table_rm = _relayout_table(table).reshape(VIEW_ROWS, EMBED_DIM)
    run = pl.kernel(
        _embed_mean_kernel,
        out_type=jax.ShapeDtypeStruct((BS * 32, OPAD), jnp.float32),
        mesh=plsc.VectorSubcoreMesh(core_axis_name="c", subcore_axis_name="s"),
        scratch_types=[
            pltpu.VMEM((IDX_PER_CHUNK,), jnp.int32),
            pltpu.VMEM((IDX_PER_CHUNK,), jnp.int32),
            pltpu.VMEM((IDX_PER_CHUNK, EMBED_DIM), jnp.float32),
            pltpu.VMEM((IDX_PER_CHUNK, EMBED_DIM), jnp.float32),
            pltpu.VMEM((CHUNK, EMBED_DIM), jnp.float32),
            pltpu.SemaphoreType.DMA,
            pltpu.SemaphoreType.DMA,
        ],
        compiler_params=pltpu.CompilerParams(use_tc_tiling_on_sc=False),
    )
    out = run(idx, table_rm)
    return _slice_out(out.reshape(BS, 32, OPAD))


# speedup vs baseline: 1.3326x; 1.3326x over previous
import jax
import jax.numpy as jnp
from jax import lax
from jax.experimental import pallas as pl
from jax.experimental.pallas import tpu as pltpu
from jax.experimental.pallas import tpu_sc as plsc

NUM_VOCAB = 1000000
EMBED_DIM = 32
BS = 4096
FEAT_LEN = 26
WORD_LEN = 20

L = 16
NC, NS = 2, 16
NW = NC * NS
ROWS = BS * FEAT_LEN
BATCH_PER_W = BS // NW
CHUNK_B = 2
CHUNK = CHUNK_B * FEAT_LEN
N_CHUNKS = BATCH_PER_W // CHUNK_B
IDX_PER_CHUNK = CHUNK * WORD_LEN
GATHER = 104
N_GATHER = IDX_PER_CHUNK // GATHER
OPAD = 128


def _embed_mean_kernel(idx_hbm, table_hbm, out_hbm,
                       idx_a, idx_b, rows_a, rows_b, out_v, sem_a, sem_b):
    wid = lax.axis_index("s") * NC + lax.axis_index("c")
    inv_n = jnp.float32(1.0 / WORD_LEN)

    def stage(c, idx_v, rows_v, sem):
        gbase = (wid * N_CHUNKS + c) * IDX_PER_CHUNK
        pltpu.sync_copy(idx_hbm.at[pl.ds(gbase, IDX_PER_CHUNK)], idx_v)
        for j in range(N_GATHER):
            pltpu.async_copy(
                table_hbm.at[idx_v.at[pl.ds(j * GATHER, GATHER)]],
                rows_v.at[pl.ds(j * GATHER, GATHER)],
                sem,
            )

    def drain(idx_v, rows_v, sem):
        for j in range(N_GATHER):
            pltpu.make_async_copy(
                table_hbm.at[idx_v.at[pl.ds(j * GATHER, GATHER)]],
                rows_v.at[pl.ds(j * GATHER, GATHER)],
                sem,
            ).wait()

    def compute(c, rows_v):
        def row_body(r, carry):
            base = r * WORD_LEN
            acc0 = rows_v[base, pl.ds(0, L)]
            acc1 = rows_v[base, pl.ds(L, L)]
            for j in range(1, WORD_LEN):
                acc0 = acc0 + rows_v[base + j, pl.ds(0, L)]
                acc1 = acc1 + rows_v[base + j, pl.ds(L, L)]
            out_v[r, pl.ds(0, L)] = acc0 * inv_n
            out_v[r, pl.ds(L, L)] = acc1 * inv_n
            return carry

        lax.fori_loop(0, CHUNK, row_body, 0, unroll=2)
        b0 = wid * BATCH_PER_W + c * CHUNK_B
        for k in range(CHUNK_B):
            pltpu.sync_copy(
                out_v.at[pl.ds(k * FEAT_LEN, FEAT_LEN)],
                out_hbm.at[pl.ds((b0 + k) * 32, FEAT_LEN), pl.ds(0, EMBED_DIM)],
            )

    stage(0, idx_a, rows_a, sem_a)

    def pair_body(i, carry):
        c0 = 2 * i
        c1 = 2 * i + 1
        stage(c1, idx_b, rows_b, sem_b)
        drain(idx_a, rows_a, sem_a)
        compute(c0, rows_a)
        stage((c0 + 2) % N_CHUNKS, idx_a, rows_a, sem_a)
        drain(idx_b, rows_b, sem_b)
        compute(c1, rows_b)
        return carry

    lax.fori_loop(0, N_CHUNKS // 2, pair_body, 0, unroll=False)
    drain(idx_a, rows_a, sem_a)


CBLK = 8192
NBLK = NUM_VOCAB // CBLK
MAIN = NBLK * CBLK
TAIL = NUM_VOCAB - MAIN
GROUPS = -(-NBLK // 4)
GRID = GROUPS + 1
OUT_ROWS = GRID * CBLK
VIEW_ROWS = OUT_ROWS * 4
TAIL_Q0 = 4 * GROUPS * CBLK - 4 * MAIN


def _transpose_kernel(t0, t1, t2, t3, tail_ref, out_ref):
    b = pl.program_id(0)

    @pl.when(b < GROUPS)
    def _main():
        stacked = jnp.concatenate([t0[...], t1[...], t2[...], t3[...]], axis=0)
        out_ref[...] = stacked.T

    @pl.when(b == GROUPS)
    def _tail():
        out_ref[0:TAIL, 0:EMBED_DIM] = tail_ref[...]


def _relayout_table(table):
    t = table.T
    tail = jax.lax.slice(table, (MAIN, 0), (NUM_VOCAB, EMBED_DIM))
    specs = [
        pl.BlockSpec((32, CBLK), lambda b, u=u: (0, jnp.minimum(4 * b + u, NBLK - 1)))
        for u in range(4)
    ]
    specs.append(pl.BlockSpec((TAIL, EMBED_DIM), lambda b: (0, 0)))
    return pl.pallas_call(
        _transpose_kernel,
        out_shape=jax.ShapeDtypeStruct((OUT_ROWS, 128), jnp.float32),
        grid=(GRID,),
        in_specs=specs,
        out_specs=pl.BlockSpec((CBLK, 128), lambda b: (b, 0)),
        compiler_params=pltpu.CompilerParams(
            dimension_semantics=("parallel",)),
    )(t, t, t, t, tail)


SLICE_B = 32


def _slice_kernel(in_ref, out_ref):
    out_ref[...] = in_ref[:, :FEAT_LEN, :EMBED_DIM]


def _slice_out(padded):
    return pl.pallas_call(
        _slice_kernel,
        out_shape=jax.ShapeDtypeStruct((BS, FEAT_LEN, EMBED_DIM), jnp.float32),
        grid=(BS // SLICE_B,),
        in_specs=[pl.BlockSpec((SLICE_B, 32, OPAD), lambda i: (i, 0, 0))],
        out_specs=pl.BlockSpec((SLICE_B, FEAT_LEN, EMBED_DIM),
                               lambda i: (i, 0, 0)),
        compiler_params=pltpu.CompilerParams(
            dimension_semantics=("parallel",)),
    )(padded)


@jax.jit
def kernel(input_text, table):
    flat = input_text.reshape(ROWS * WORD_LEN).astype(jnp.int32)
    m = flat // CBLK
    q_main = 4 * ((m // 4) * CBLK + (flat % CBLK)) + (m % 4)
    idx = jnp.where(flat >= MAIN, 4 * flat + TAIL_Q0, q_main)
    table_rm = _relayout_table(table).reshape(VIEW_ROWS, EMBED_DIM)
    run = pl.kernel(
        _embed_mean_kernel,
        out_type=jax.ShapeDtypeStruct((BS * 32, OPAD), jnp.float32),
        mesh=plsc.VectorSubcoreMesh(core_axis_name="c", subcore_axis_name="s"),
        scratch_types=[
            pltpu.VMEM((IDX_PER_CHUNK,), jnp.int32),
            pltpu.VMEM((IDX_PER_CHUNK,), jnp.int32),
            pltpu.VMEM((IDX_PER_CHUNK, EMBED_DIM), jnp.float32),
            pltpu.VMEM((IDX_PER_CHUNK, EMBED_DIM), jnp.float32),
            pltpu.VMEM((CHUNK, EMBED_DIM), jnp.float32),
            pltpu.SemaphoreType.DMA,
            pltpu.SemaphoreType.DMA,
        ],
        compiler_params=pltpu.CompilerParams(use_tc_tiling_on_sc=False),
    )
    out = run(idx, table_rm)
    out = out.reshape(BS, 32, OPAD)
    return lax.slice(out, (0, 0, 0), (BS, FEAT_LEN, EMBED_DIM))
